# Initial kernel scaffold; baseline (speedup 1.0000x reference)
import jax
import jax.numpy as jnp
from jax.experimental import pallas as pl


def kernel(queue, features, ptr):
    raise NotImplementedError("write your pallas kernel here")


# speedup vs baseline: 2.2555x; 2.2555x over previous
import jax
import jax.numpy as jnp
from jax.experimental import pallas as pl
from jax.experimental.pallas import tpu as pltpu

_CAP = 1000000
_N = 16384
_D = 64
_BLK = 10000
_NB = _CAP // _BLK
_NW = (_N + _BLK - 1) // _BLK + 1


def _body(s_ref, q_ref, f_ref, o_ref):
    ptr = s_ref[0]
    k = pl.program_id(0)
    rows = k * _BLK + jax.lax.broadcasted_iota(jnp.int32, (_BLK, 1), 0)
    d = rows - ptr
    off = jnp.where(d < 0, d + _CAP, d)
    mask = off < _N
    o_ref[...] = jnp.where(mask, f_ref[...], q_ref[...])


def _f_index(k, s_ref):
    p0 = s_ref[0] // _BLK
    j = k - p0
    j = jnp.where(j < 0, j + _NB, j)
    return (jnp.where(j < _NW, j, 0), 0)


def kernel(queue, features, ptr):
    ptr = jnp.asarray(ptr, jnp.int32)
    a = ptr % _BLK
    fshift = jax.lax.dynamic_update_slice(
        jnp.zeros((_NW * _BLK, _D), jnp.float32), features, (a, 0))
    grid_spec = pltpu.PrefetchScalarGridSpec(
        num_scalar_prefetch=1,
        grid=(_NB,),
        in_specs=[
            pl.BlockSpec((_BLK, _D), lambda k, s: (k, 0)),
            pl.BlockSpec((_BLK, _D), _f_index),
        ],
        out_specs=pl.BlockSpec((_BLK, _D), lambda k, s: (k, 0)),
    )
    return pl.pallas_call(
        _body,
        grid_spec=grid_spec,
        out_shape=jax.ShapeDtypeStruct((_CAP, _D), jnp.float32),
    )(ptr.reshape(1), queue, fshift)
